# Initial kernel scaffold; baseline (speedup 1.0000x reference)
import jax
import jax.numpy as jnp
from jax.experimental import pallas as pl


def kernel(x, edge_index, W1, b1, W2, b2):
    raise NotImplementedError("write your pallas kernel here")


# speedup vs baseline: 28.4977x; 28.4977x over previous
import functools

import jax
import jax.numpy as jnp
from jax import lax
from jax.experimental import pallas as pl
from jax.experimental.pallas import tpu as pltpu
from jax.experimental.pallas import tpu_sc as plsc

N = 100000
NPAD = 102400
RPT = NPAD // 16
E = 1600000
LANES = 128
J = 4
G = 100
NW = 32
RPW = G * J
EP = NW * RPW * LANES
EROWS = EP // LANES

_mesh = lambda: plsc.VectorSubcoreMesh(core_axis_name="c", subcore_axis_name="s")
_SC_PARAMS = pltpu.CompilerParams(use_tc_tiling_on_sc=False)


JL = J * LANES


def _make_edge_agg(ch):
    @functools.partial(
        pl.kernel,
        out_type=jax.ShapeDtypeStruct((2, NPAD, ch), jnp.float32),
        mesh=_mesh(),
        scratch_types=[
            pltpu.VMEM((2, J, LANES), jnp.int32),
            pltpu.VMEM((J, LANES), jnp.int32),
            pltpu.VMEM((2, JL, ch), jnp.float32),
            pltpu.VMEM_SHARED((NPAD, ch), jnp.float32),
            pltpu.SemaphoreType.DMA,
        ],
        compiler_params=_SC_PARAMS,
    )
    def agg(table, srcr, dstr, zrows, out, idx_s, idx_d, msg, acc, gsem):
        c = lax.axis_index("c")
        s = lax.axis_index("s")
        wid = s * 2 + c
        base = s * RPT
        pltpu.sync_copy(zrows, acc.at[pl.ds(base, RPT)])
        plsc.subcore_barrier()

        def fire(g, b):
            row0 = wid * RPW + g * J
            pltpu.sync_copy(srcr.at[pl.ds(row0, J)], idx_s.at[b])

            def go(j, carry):
                pltpu.async_copy(
                    table.at[idx_s.at[b, j]],
                    msg.at[b, pl.ds(j * LANES, LANES)], gsem)
                return carry

            lax.fori_loop(0, J, go, 0)

        def scatter(g, b):
            row0 = wid * RPW + g * J
            pltpu.sync_copy(dstr.at[pl.ds(row0, J)], idx_d)
            pltpu.make_async_copy(zrows.at[pl.ds(0, JL)], msg.at[b], gsem).wait()

            def go(j, carry):
                pltpu.sync_copy(msg.at[b, pl.ds(j * LANES, LANES)],
                                acc.at[idx_d.at[j]], add=True)
                return carry

            lax.fori_loop(0, J, go, 0)

        fire(0, 0)

        def outer(g, carry):
            b = lax.rem(g, 2)
            fire(g, b)
            scatter(g - 1, 1 - b)
            return carry

        lax.fori_loop(1, G, outer, 0)
        scatter(G - 1, (G - 1) % 2)
        plsc.subcore_barrier()
        pltpu.sync_copy(acc.at[pl.ds(base, RPT)], out.at[c, pl.ds(base, RPT)])

    return agg


def _make_deg():
    @functools.partial(
        pl.kernel,
        out_type=jax.ShapeDtypeStruct((2, NPAD, 8), jnp.float32),
        mesh=_mesh(),
        scratch_types=[
            pltpu.VMEM((J, LANES), jnp.int32),
            pltpu.VMEM((LANES, 8), jnp.float32),
            pltpu.VMEM_SHARED((NPAD, 8), jnp.float32),
        ],
        compiler_params=_SC_PARAMS,
    )
    def deg(dstr, zrows, ones, out, idx_d, onev, acc):
        c = lax.axis_index("c")
        s = lax.axis_index("s")
        wid = s * 2 + c
        base = s * RPT
        pltpu.sync_copy(ones, onev)
        pltpu.sync_copy(zrows, acc.at[pl.ds(base, RPT)])
        plsc.subcore_barrier()

        def outer(g, carry):
            row0 = wid * RPW + g * J
            pltpu.sync_copy(dstr.at[pl.ds(row0, J)], idx_d)

            def inner(j, carry2):
                pltpu.sync_copy(onev, acc.at[idx_d.at[j]], add=True)
                return carry2

            return lax.fori_loop(0, J, inner, carry)

        lax.fori_loop(0, G, outer, 0)
        plsc.subcore_barrier()
        pltpu.sync_copy(acc.at[pl.ds(base, RPT)], out.at[c, pl.ds(base, RPT)])

    return deg


_ROWS_BLK = 4000
_GRID = N // _ROWS_BLK


def _prep_body(d0, d1, x, dinv, xs):
    deg = d0[...] + d1[...] + 1.0
    di = lax.rsqrt(deg)
    dinv[...] = di
    xs[...] = x[...] * di


def _mid_body(p0, p1, xs, dinv, w1, b1, w2, h2s):
    agg = (p0[...] + p1[...] + xs[...]) * dinv[...]
    t = jnp.dot(agg, w1[...], preferred_element_type=jnp.float32) + b1[...]
    h = jnp.maximum(t, 0.0)
    h2s[...] = jnp.dot(h, w2[...], preferred_element_type=jnp.float32) * dinv[...]


def _post_body(q0, q1, h2s, dinv, b2, out):
    out[...] = (q0[...] + q1[...] + h2s[...]) * dinv[...] + b2[...]


def _rows_spec(ch):
    return pl.BlockSpec((_ROWS_BLK, ch), lambda i: (i, 0))


def _full_spec(r, c):
    return pl.BlockSpec((r, c), lambda i: (0, 0))


def kernel(x, edge_index, W1, b1, W2, b2):
    src = edge_index[0].astype(jnp.int32)
    dst = edge_index[1].astype(jnp.int32)
    pad = EP - E
    srcr = jnp.concatenate([src, jnp.zeros((pad,), jnp.int32)]).reshape(EROWS, LANES)
    dstr = jnp.concatenate([dst, jnp.full((pad,), N, jnp.int32)]).reshape(EROWS, LANES)

    degp = _make_deg()(
        dstr, jnp.zeros((RPT, 8), jnp.float32), jnp.ones((LANES, 8), jnp.float32)
    )
    d0 = degp[0, :N, 0:1]
    d1 = degp[1, :N, 0:1]

    dinv, xs = pl.pallas_call(
        _prep_body,
        grid=(_GRID,),
        in_specs=[_rows_spec(1), _rows_spec(1), _rows_spec(8)],
        out_specs=[_rows_spec(1), _rows_spec(8)],
        out_shape=[
            jax.ShapeDtypeStruct((N, 1), jnp.float32),
            jax.ShapeDtypeStruct((N, 8), jnp.float32),
        ],
    )(d0, d1, x)

    p = _make_edge_agg(8)(xs, srcr, dstr, jnp.zeros((RPT, 8), jnp.float32))

    h2s = pl.pallas_call(
        _mid_body,
        grid=(_GRID,),
        in_specs=[
            _rows_spec(8), _rows_spec(8), _rows_spec(8), _rows_spec(1),
            _full_spec(8, 64), _full_spec(1, 64), _full_spec(64, 16),
        ],
        out_specs=_rows_spec(16),
        out_shape=jax.ShapeDtypeStruct((N, 16), jnp.float32),
    )(p[0, :N], p[1, :N], xs, dinv, W1, b1.reshape(1, 64), W2)

    q = _make_edge_agg(16)(h2s, srcr, dstr, jnp.zeros((RPT, 16), jnp.float32))

    out = pl.pallas_call(
        _post_body,
        grid=(_GRID,),
        in_specs=[
            _rows_spec(16), _rows_spec(16), _rows_spec(16), _rows_spec(1),
            _full_spec(1, 16),
        ],
        out_specs=_rows_spec(16),
        out_shape=jax.ShapeDtypeStruct((N, 16), jnp.float32),
    )(q[0, :N], q[1, :N], h2s, dinv, b2.reshape(1, 16))

    return out
